# baseline (speedup 1.0000x reference)
import jax
import jax.numpy as jnp
from jax.experimental import pallas as pl
from jax.experimental.pallas import tpu as pltpu

_O_CLS, _O_REG, _O_DIR = 18, 42, 12
_OFF_CLS, _OFF_REG, _OFF_DIR = 0, 24, 72
_PACKED = 84
_H_BLK = 8


def _fused_heads_kernel(x_ref, wt_ref, bias_ref, cls_ref, reg_ref, dir_ref):
    acc = jax.lax.dot_general(
        wt_ref[:], x_ref[0].astype(jnp.bfloat16),
        (((1,), (0,)), ((), ())),
        preferred_element_type=jnp.float32,
    )
    acc = acc + bias_ref[:]
    cls_ref[0] = acc[_OFF_CLS:_OFF_CLS + _O_CLS]
    reg_ref[0] = acc[_OFF_REG:_OFF_REG + _O_REG]
    dir_ref[0] = acc[_OFF_DIR:_OFF_DIR + _O_DIR]


def kernel(x, W_cls, b_cls, W_reg, b_reg, W_dir, b_dir):
    B, C, H, W = x.shape

    wt = jnp.concatenate(
        [
            W_cls.T.astype(jnp.bfloat16),
            jnp.zeros((_OFF_REG - _O_CLS, C), dtype=jnp.bfloat16),
            W_reg.T.astype(jnp.bfloat16),
            jnp.zeros((_OFF_DIR - _OFF_REG - _O_REG, C), dtype=jnp.bfloat16),
            W_dir.T.astype(jnp.bfloat16),
        ],
        axis=0,
    )
    bias = jnp.concatenate(
        [
            b_cls,
            jnp.zeros((_OFF_REG - _O_CLS,), dtype=x.dtype),
            b_reg,
            jnp.zeros((_OFF_DIR - _OFF_REG - _O_REG,), dtype=x.dtype),
            b_dir,
        ]
    ).reshape(_PACKED, 1, 1)

    nh = pl.cdiv(H, _H_BLK)
    return pl.pallas_call(
        _fused_heads_kernel,
        grid=(B, nh),
        in_specs=[
            pl.BlockSpec((1, C, _H_BLK, W), lambda b, h: (b, 0, h, 0)),
            pl.BlockSpec((_PACKED, C), lambda b, h: (0, 0)),
            pl.BlockSpec((_PACKED, 1, 1), lambda b, h: (0, 0, 0)),
        ],
        out_specs=[
            pl.BlockSpec((1, _O_CLS, _H_BLK, W), lambda b, h: (b, 0, h, 0)),
            pl.BlockSpec((1, _O_REG, _H_BLK, W), lambda b, h: (b, 0, h, 0)),
            pl.BlockSpec((1, _O_DIR, _H_BLK, W), lambda b, h: (b, 0, h, 0)),
        ],
        out_shape=[
            jax.ShapeDtypeStruct((B, _O_CLS, H, W), jnp.float32),
            jax.ShapeDtypeStruct((B, _O_REG, H, W), jnp.float32),
            jax.ShapeDtypeStruct((B, _O_DIR, H, W), jnp.float32),
        ],
        compiler_params=pltpu.CompilerParams(
            dimension_semantics=("parallel", "arbitrary"),
        ),
    )(x, wt, bias)


# speedup vs baseline: 1.0890x; 1.0890x over previous
import jax
import jax.numpy as jnp
from jax.experimental import pallas as pl
from jax.experimental.pallas import tpu as pltpu

_O_CLS, _O_REG, _O_DIR = 18, 42, 12
_OFF_CLS, _OFF_REG, _OFF_DIR = 0, 24, 72
_PACKED = 84
_H_BLK = 32


def _fused_heads_kernel(x_ref, wt_ref, bias_ref, cls_ref, reg_ref, dir_ref):
    acc = jax.lax.dot_general(
        wt_ref[:], x_ref[0].astype(jnp.bfloat16),
        (((1,), (0,)), ((), ())),
        preferred_element_type=jnp.float32,
    )
    acc = acc + bias_ref[:]
    cls_ref[0] = acc[_OFF_CLS:_OFF_CLS + _O_CLS]
    reg_ref[0] = acc[_OFF_REG:_OFF_REG + _O_REG]
    dir_ref[0] = acc[_OFF_DIR:_OFF_DIR + _O_DIR]


def kernel(x, W_cls, b_cls, W_reg, b_reg, W_dir, b_dir):
    B, C, H, W = x.shape

    wt = jnp.concatenate(
        [
            W_cls.T.astype(jnp.bfloat16),
            jnp.zeros((_OFF_REG - _O_CLS, C), dtype=jnp.bfloat16),
            W_reg.T.astype(jnp.bfloat16),
            jnp.zeros((_OFF_DIR - _OFF_REG - _O_REG, C), dtype=jnp.bfloat16),
            W_dir.T.astype(jnp.bfloat16),
        ],
        axis=0,
    )
    bias = jnp.concatenate(
        [
            b_cls,
            jnp.zeros((_OFF_REG - _O_CLS,), dtype=x.dtype),
            b_reg,
            jnp.zeros((_OFF_DIR - _OFF_REG - _O_REG,), dtype=x.dtype),
            b_dir,
        ]
    ).reshape(_PACKED, 1, 1)

    nh = pl.cdiv(H, _H_BLK)
    return pl.pallas_call(
        _fused_heads_kernel,
        grid=(B, nh),
        in_specs=[
            pl.BlockSpec((1, C, _H_BLK, W), lambda b, h: (b, 0, h, 0)),
            pl.BlockSpec((_PACKED, C), lambda b, h: (0, 0)),
            pl.BlockSpec((_PACKED, 1, 1), lambda b, h: (0, 0, 0)),
        ],
        out_specs=[
            pl.BlockSpec((1, _O_CLS, _H_BLK, W), lambda b, h: (b, 0, h, 0)),
            pl.BlockSpec((1, _O_REG, _H_BLK, W), lambda b, h: (b, 0, h, 0)),
            pl.BlockSpec((1, _O_DIR, _H_BLK, W), lambda b, h: (b, 0, h, 0)),
        ],
        out_shape=[
            jax.ShapeDtypeStruct((B, _O_CLS, H, W), jnp.float32),
            jax.ShapeDtypeStruct((B, _O_REG, H, W), jnp.float32),
            jax.ShapeDtypeStruct((B, _O_DIR, H, W), jnp.float32),
        ],
        compiler_params=pltpu.CompilerParams(
            dimension_semantics=("parallel", "arbitrary"),
        ),
    )(x, wt, bias)
